# Initial kernel scaffold; baseline (speedup 1.0000x reference)
import jax
import jax.numpy as jnp
from jax.experimental import pallas as pl


def kernel(encoded, span_starts, span_lengths, pair_head, pair_tail, W, b):
    raise NotImplementedError("write your pallas kernel here")


# speedup vs baseline: 24.8370x; 24.8370x over previous
import functools

import jax
import jax.numpy as jnp
from jax import lax
from jax.experimental import pallas as pl
from jax.experimental.pallas import tpu as pltpu

B, T, D, S, P, R = 8, 2048, 256, 64, 2048, 16
SPAN_WIN = 32


def _pool_project_gather_kernel(starts_ref, lens_ref, encoded_ref, head_ref,
                                tail_ref, w_ref, b_ref, out_ref, pooled_ref):
    bidx = pl.program_id(0)
    neg = jnp.finfo(jnp.float32).min

    def body(s, _):
        start = starts_ref[bidx, s]
        ln = lens_ref[bidx, s]
        base = (start // 8) * 8
        off = start - base
        rows = encoded_ref[0, pl.ds(base, SPAN_WIN + 8), :]
        row_id = lax.broadcasted_iota(jnp.int32, (SPAN_WIN + 8, D), 0)
        masked = jnp.where((row_id >= off) & (row_id < off + ln), rows, neg)
        pooled_ref[pl.ds(s, 1), :] = jnp.max(masked, axis=0, keepdims=True)
        return 0

    lax.fori_loop(0, S, body, 0)

    pooled = pooled_ref[...]
    h = jnp.dot(pooled, w_ref[:D, :], preferred_element_type=jnp.float32)
    h = h + b_ref[...][None, :]
    t = jnp.dot(pooled, w_ref[D:, :], preferred_element_type=jnp.float32)

    span_id = lax.broadcasted_iota(jnp.int32, (P, S), 1)
    oh_head = (head_ref[0, 0, :][:, None] == span_id).astype(jnp.float32)
    oh_tail = (tail_ref[0, 0, :][:, None] == span_id).astype(jnp.float32)
    scores = (jnp.dot(oh_head, h, preferred_element_type=jnp.float32)
              + jnp.dot(oh_tail, t, preferred_element_type=jnp.float32))
    out_ref[0, :, :] = scores


def kernel(encoded, span_starts, span_lengths, pair_head, pair_tail, W, b):
    head3 = pair_head.astype(jnp.int32).reshape(B, 1, P)
    tail3 = pair_tail.astype(jnp.int32).reshape(B, 1, P)
    grid_spec = pltpu.PrefetchScalarGridSpec(
        num_scalar_prefetch=2,
        grid=(B,),
        in_specs=[
            pl.BlockSpec((1, T, D), lambda b_, *_: (b_, 0, 0)),
            pl.BlockSpec((1, 1, P), lambda b_, *_: (b_, 0, 0)),
            pl.BlockSpec((1, 1, P), lambda b_, *_: (b_, 0, 0)),
            pl.BlockSpec((2 * D, R), lambda b_, *_: (0, 0)),
            pl.BlockSpec((R,), lambda b_, *_: (0,)),
        ],
        out_specs=pl.BlockSpec((1, P, R), lambda b_, *_: (b_, 0, 0)),
        scratch_shapes=[pltpu.VMEM((S, D), jnp.float32)],
    )
    return pl.pallas_call(
        _pool_project_gather_kernel,
        grid_spec=grid_spec,
        out_shape=jax.ShapeDtypeStruct((B, P, R), jnp.float32),
    )(span_starts.astype(jnp.int32), span_lengths.astype(jnp.int32),
      encoded, head3, tail3, W, b)
